# baseline (speedup 1.0000x reference)
import functools

import jax
import jax.numpy as jnp
import numpy as np
from jax.experimental import pallas as pl
from jax.experimental.pallas import tpu as pltpu
from jax.experimental.pallas import tpu_sc as plsc

_TINY = np.float32(np.finfo(np.float32).tiny)
_IMAX = np.int32(np.iinfo(np.int32).max)

B = 64
VOCAB = 1000000
TILE = 4096
NT = -(-VOCAB // TILE)
SC_NCH = 102
J0 = NT - SC_NCH
SC_COL0 = J0 * TILE
SC_COLS = SC_NCH * TILE
CH = TILE
UNROLL = 8


def _threefry_bits(flat_i):
    k0 = jnp.uint32(0)
    k1 = jnp.uint32(42)
    ks2 = jnp.uint32(0 ^ 42 ^ 0x1BD11BDA)

    def rotl(x, d):
        return (x << jnp.uint32(d)) | (x >> jnp.uint32(32 - d))

    x0 = flat_i + k1
    x1 = rotl(x0, 13) ^ x0
    rots = ((13, 15, 26, 6), (17, 29, 16, 24))
    sched = ((k1, ks2), (ks2, k0), (k0, k1), (k1, ks2), (ks2, k0))
    for j in range(5):
        for r in rots[j % 2][(1 if j == 0 else 0):]:
            x0 = x0 + x1
            x1 = rotl(x1, r)
            x1 = x0 ^ x1
        a, b = sched[j]
        x0 = x0 + a
        x1 = x1 + b + jnp.uint32(j + 1)
    return x0 ^ x1


def _bits_to_gumbel(bits):
    f = jax.lax.bitcast_convert_type(
        (bits >> jnp.uint32(9)) | jnp.uint32(0x3F800000), jnp.float32)
    f = f - jnp.float32(1.0)
    u = jnp.maximum(_TINY, f * (jnp.float32(1.0) - _TINY) + _TINY)
    return -jnp.log(-jnp.log(u))


def _sc_bits_kernel(out_ref, buf):
    c = jax.lax.axis_index("c")
    s = jax.lax.axis_index("s")
    w = s * 2 + c
    for rr in range(2):
        r = w * 2 + rr
        rowbase = r * VOCAB + SC_COL0

        def chunk_body(q, _):
            cb = rowbase + q * CH

            def vec_body(i, _):
                for uu in range(UNROLL):
                    off = i * (16 * UNROLL) + uu * 16
                    cnt = (cb + off + jax.lax.iota(jnp.int32, 16)).astype(
                        jnp.uint32)
                    buf[pl.ds(off, 16)] = _threefry_bits(cnt)
                return 0

            jax.lax.fori_loop(0, CH // (16 * UNROLL), vec_body, 0)
            pltpu.sync_copy(buf, out_ref.at[r, pl.ds(q * CH, CH)])
            return 0

        jax.lax.fori_loop(0, SC_NCH, chunk_body, 0)


def _pass_sums(x_ref, sum_ref, *, tile, vocab):
    j = pl.program_id(0)
    x = x_ref[...]
    b = x.shape[0]
    col = jax.lax.broadcasted_iota(jnp.int32, (b, tile), 1) + j * tile
    sum_ref[0, 0, :] = jnp.sum(jnp.where(col < vocab, jnp.exp(x), 0.0), axis=1)


def _pass_inv(sum_ref, inv_ref):
    inv_ref[0, :] = jnp.float32(1.0) / jnp.sum(sum_ref[:, 0, :], axis=0)


def _finish(x, col, vocab, g, inv, probs_ref, max_ref, idx_ref):
    probs_ref[...] = jnp.exp(x) * inv
    t = jnp.where(col < vocab, g + x, -jnp.inf)
    m = jnp.max(t, axis=1)
    cand = jnp.where(t == m[:, None], col, _IMAX)
    max_ref[0, 0, :] = m
    idx_ref[0, 0, :] = jnp.min(cand, axis=1)


def _main_a(x_ref, inv_ref, f0_ref, probs_ref, max_ref, idx_ref,
            *, tile, vocab):
    j = pl.program_id(0)
    x = x_ref[...]
    b = x.shape[0]
    flat = f0_ref[...] + jnp.uint32(j * tile)
    g = _bits_to_gumbel(_threefry_bits(flat))
    probs_ref[...] = jnp.exp(x) * inv_ref[...]
    lcol = jax.lax.broadcasted_iota(jnp.int32, (b, tile), 1)
    t = jnp.where(lcol < (vocab - j * tile), g + x, -jnp.inf)
    m = jnp.max(t, axis=1)
    cand = jnp.where(t == m[:, None], lcol, _IMAX)
    max_ref[j, 0, :] = m
    idx_ref[j, 0, :] = jnp.min(cand, axis=1) + j * tile


def _main_b(x_ref, bits_ref, inv_ref, pin_ref, probs_ref, max_ref, idx_ref,
            *, tile, vocab, col0):
    j = pl.program_id(0)
    x = x_ref[...]
    b = x.shape[0]
    g = _bits_to_gumbel(bits_ref[...])
    probs_ref[...] = jnp.exp(x) * inv_ref[...]
    lcol = jax.lax.broadcasted_iota(jnp.int32, (b, tile), 1)
    t = jnp.where(lcol < (vocab - col0 - j * tile), g + x, -jnp.inf)
    m = jnp.max(t, axis=1)
    cand = jnp.where(t == m[:, None], lcol, _IMAX)
    max_ref[j, 0, :] = m
    idx_ref[j, 0, :] = jnp.min(cand, axis=1) + (col0 + j * tile)


def _pass_idx(max_ref, idx_ref, out_idx_ref):
    m = jnp.max(max_ref[:, 0, :], axis=0)
    cand = jnp.where(max_ref[:, 0, :] == m[None, :], idx_ref[:, 0, :], _IMAX)
    out_idx_ref[0, :] = jnp.min(cand, axis=0)


def kernel(logits):
    b, vocab = logits.shape

    sc_bits = pl.kernel(
        _sc_bits_kernel,
        out_type=jax.ShapeDtypeStruct((B, SC_COLS), jnp.uint32),
        scratch_types=[pltpu.VMEM((CH,), jnp.uint32)],
        mesh=plsc.VectorSubcoreMesh(core_axis_name="c", subcore_axis_name="s"),
    )()

    sums = pl.pallas_call(
        functools.partial(_pass_sums, tile=TILE, vocab=vocab),
        grid=(NT,),
        in_specs=[pl.BlockSpec((B, TILE), lambda j: (0, j))],
        out_specs=pl.BlockSpec((1, 1, B), lambda j: (j, 0, 0)),
        out_shape=jax.ShapeDtypeStruct((NT, 1, B), jnp.float32),
        compiler_params=pltpu.CompilerParams(
            dimension_semantics=("parallel",)),
    )(logits)

    inv = pl.pallas_call(
        _pass_inv,
        out_shape=jax.ShapeDtypeStruct((1, b), jnp.float32),
    )(sums)
    inv_col = inv.reshape(b, 1)

    flat0 = (jnp.arange(b, dtype=jnp.uint32)[:, None] * jnp.uint32(vocab)
             + jnp.arange(TILE, dtype=jnp.uint32)[None, :])

    probs_a, maxs_a, idxs_a = pl.pallas_call(
        functools.partial(_main_a, tile=TILE, vocab=vocab),
        grid=(J0,),
        in_specs=[
            pl.BlockSpec((B, TILE), lambda j: (0, j)),
            pl.BlockSpec((B, 1), lambda j: (0, 0)),
            pl.BlockSpec((B, TILE), lambda j: (0, 0)),
        ],
        out_specs=[
            pl.BlockSpec((B, TILE), lambda j: (0, j)),
            pl.BlockSpec((J0, 1, B), lambda j: (0, 0, 0)),
            pl.BlockSpec((J0, 1, B), lambda j: (0, 0, 0)),
        ],
        out_shape=[
            jax.ShapeDtypeStruct((b, vocab), jnp.float32),
            jax.ShapeDtypeStruct((J0, 1, B), jnp.float32),
            jax.ShapeDtypeStruct((J0, 1, B), jnp.int32),
        ],
        compiler_params=pltpu.CompilerParams(
            dimension_semantics=("arbitrary",)),
    )(logits, inv_col, flat0)

    probs, maxs_b, idxs_b = pl.pallas_call(
        functools.partial(_main_b, tile=TILE, vocab=vocab, col0=SC_COL0),
        grid=(SC_NCH,),
        in_specs=[
            pl.BlockSpec((B, TILE), lambda j: (0, j + J0)),
            pl.BlockSpec((B, TILE), lambda j: (0, j)),
            pl.BlockSpec((B, 1), lambda j: (0, 0)),
            pl.BlockSpec((8, 128), lambda j: (0, 0)),
        ],
        out_specs=[
            pl.BlockSpec((B, TILE), lambda j: (0, j + J0)),
            pl.BlockSpec((SC_NCH, 1, B), lambda j: (0, 0, 0)),
            pl.BlockSpec((SC_NCH, 1, B), lambda j: (0, 0, 0)),
        ],
        out_shape=[
            jax.ShapeDtypeStruct((b, vocab), jnp.float32),
            jax.ShapeDtypeStruct((SC_NCH, 1, B), jnp.float32),
            jax.ShapeDtypeStruct((SC_NCH, 1, B), jnp.int32),
        ],
        input_output_aliases={3: 0},
        compiler_params=pltpu.CompilerParams(
            dimension_semantics=("arbitrary",)),
    )(logits, sc_bits, inv_col, probs_a)

    maxs = jnp.concatenate([maxs_a, maxs_b], axis=0)
    idxs = jnp.concatenate([idxs_a, idxs_b], axis=0)

    idx = pl.pallas_call(
        _pass_idx,
        out_shape=jax.ShapeDtypeStruct((1, b), jnp.int32),
    )(maxs, idxs)

    idx_next = idx.reshape(b, 1).astype(jnp.int64)
    return (probs, idx_next)


# speedup vs baseline: 1.1226x; 1.1226x over previous
import functools

import jax
import jax.numpy as jnp
import numpy as np
from jax.experimental import pallas as pl
from jax.experimental.pallas import tpu as pltpu
from jax.experimental.pallas import tpu_sc as plsc

_TINY = np.float32(np.finfo(np.float32).tiny)
_IMAX = np.int32(np.iinfo(np.int32).max)

B = 64
VOCAB = 1000000
TILE = 4096
NT = -(-VOCAB // TILE)
J0 = 156
SC_COL0 = J0 * TILE
SC_NCH = 90
SC_COLS = SC_NCH * TILE
TILE_B = 2 * TILE
NB = SC_COLS // TILE_B
CH = TILE
UNROLL = 8


def _threefry_bits(flat_i):
    k0 = jnp.uint32(0)
    k1 = jnp.uint32(42)
    ks2 = jnp.uint32(0 ^ 42 ^ 0x1BD11BDA)

    def rotl(x, d):
        return (x << jnp.uint32(d)) | (x >> jnp.uint32(32 - d))

    x0 = flat_i + k1
    x1 = rotl(x0, 13) ^ x0
    rots = ((13, 15, 26, 6), (17, 29, 16, 24))
    sched = ((k1, ks2), (ks2, k0), (k0, k1), (k1, ks2), (ks2, k0))
    for j in range(5):
        for r in rots[j % 2][(1 if j == 0 else 0):]:
            x0 = x0 + x1
            x1 = rotl(x1, r)
            x1 = x0 ^ x1
        a, b = sched[j]
        x0 = x0 + a
        x1 = x1 + b + jnp.uint32(j + 1)
    return x0 ^ x1


def _bits_to_gumbel(bits):
    f = jax.lax.bitcast_convert_type(
        (bits >> jnp.uint32(9)) | jnp.uint32(0x3F800000), jnp.float32)
    f = f - jnp.float32(1.0)
    u = jnp.maximum(_TINY, f * (jnp.float32(1.0) - _TINY) + _TINY)
    return -jnp.log(-jnp.log(u))


def _sc_bits_kernel(out_ref, buf):
    c = jax.lax.axis_index("c")
    s = jax.lax.axis_index("s")
    w = s * 2 + c
    for rr in range(2):
        r = w * 2 + rr
        rowbase = r * VOCAB + SC_COL0

        def chunk_body(q, _):
            cb = rowbase + q * CH

            def vec_body(i, _):
                for uu in range(UNROLL):
                    off = i * (16 * UNROLL) + uu * 16
                    cnt = (cb + off + jax.lax.iota(jnp.int32, 16)).astype(
                        jnp.uint32)
                    buf[pl.ds(off, 16)] = _threefry_bits(cnt)
                return 0

            jax.lax.fori_loop(0, CH // (16 * UNROLL), vec_body, 0)
            pltpu.sync_copy(buf, out_ref.at[r, pl.ds(q * CH, CH)])
            return 0

        jax.lax.fori_loop(0, SC_NCH, chunk_body, 0)


def _pass_sums(x_ref, sum_ref, *, tile, vocab):
    j = pl.program_id(0)
    x = x_ref[...]
    b = x.shape[0]
    col = jax.lax.broadcasted_iota(jnp.int32, (b, tile), 1) + j * tile
    sum_ref[0, 0, :] = jnp.sum(jnp.where(col < vocab, jnp.exp(x), 0.0), axis=1)


def _pass_inv(sum_ref, inv_ref):
    inv_ref[0, :] = jnp.float32(1.0) / jnp.sum(sum_ref[:, 0, :], axis=0)


def _finish(x, col, vocab, g, inv, probs_ref, max_ref, idx_ref):
    probs_ref[...] = jnp.exp(x) * inv
    t = jnp.where(col < vocab, g + x, -jnp.inf)
    m = jnp.max(t, axis=1)
    cand = jnp.where(t == m[:, None], col, _IMAX)
    max_ref[0, 0, :] = m
    idx_ref[0, 0, :] = jnp.min(cand, axis=1)


def _main_a(x_ref, inv_ref, f0_ref, probs_ref, max_ref, idx_ref,
            *, tile, vocab):
    j = pl.program_id(0)
    x = x_ref[...]
    b = x.shape[0]
    flat = f0_ref[...] + jnp.uint32(j * tile)
    g = _bits_to_gumbel(_threefry_bits(flat))
    probs_ref[...] = jnp.exp(x) * inv_ref[...]
    lcol = jax.lax.broadcasted_iota(jnp.int32, (b, tile), 1)
    t = jnp.where(lcol < (vocab - j * tile), g + x, -jnp.inf)
    m = jnp.max(t, axis=1)
    cand = jnp.where(t == m[:, None], lcol, _IMAX)
    max_ref[j, 0, :] = m
    idx_ref[j, 0, :] = jnp.min(cand, axis=1) + j * tile


def _main_b(x_ref, bits_ref, inv_ref, pin_ref, probs_ref, max_ref, idx_ref,
            *, tile, vocab, col0):
    j = pl.program_id(0)
    x = x_ref[...]
    b = x.shape[0]
    g = _bits_to_gumbel(bits_ref[...])
    probs_ref[...] = jnp.exp(x) * inv_ref[...]
    lcol = jax.lax.broadcasted_iota(jnp.int32, (b, tile), 1)
    t = jnp.where(lcol < (vocab - col0 - j * tile), g + x, -jnp.inf)
    m = jnp.max(t, axis=1)
    cand = jnp.where(t == m[:, None], lcol, _IMAX)
    max_ref[j, 0, :] = m
    idx_ref[j, 0, :] = jnp.min(cand, axis=1) + (col0 + j * tile)


def _pass_idx(max_ref, idx_ref, out_idx_ref):
    m = jnp.max(max_ref[:, 0, :], axis=0)
    cand = jnp.where(max_ref[:, 0, :] == m[None, :], idx_ref[:, 0, :], _IMAX)
    out_idx_ref[0, :] = jnp.min(cand, axis=0)


def kernel(logits):
    b, vocab = logits.shape

    sc_bits = pl.kernel(
        _sc_bits_kernel,
        out_type=jax.ShapeDtypeStruct((B, SC_COLS), jnp.uint32),
        scratch_types=[pltpu.VMEM((CH,), jnp.uint32)],
        mesh=plsc.VectorSubcoreMesh(core_axis_name="c", subcore_axis_name="s"),
    )()

    sums = pl.pallas_call(
        functools.partial(_pass_sums, tile=TILE, vocab=vocab),
        grid=(NT,),
        in_specs=[pl.BlockSpec((B, TILE), lambda j: (0, j))],
        out_specs=pl.BlockSpec((1, 1, B), lambda j: (j, 0, 0)),
        out_shape=jax.ShapeDtypeStruct((NT, 1, B), jnp.float32),
        compiler_params=pltpu.CompilerParams(
            dimension_semantics=("parallel",)),
    )(logits)

    inv = pl.pallas_call(
        _pass_inv,
        out_shape=jax.ShapeDtypeStruct((1, b), jnp.float32),
    )(sums)
    inv_col = inv.reshape(b, 1)

    flat0 = (jnp.arange(b, dtype=jnp.uint32)[:, None] * jnp.uint32(vocab)
             + jnp.arange(TILE, dtype=jnp.uint32)[None, :])

    probs_a, maxs_a, idxs_a = pl.pallas_call(
        functools.partial(_main_a, tile=TILE, vocab=vocab),
        grid=(J0,),
        in_specs=[
            pl.BlockSpec((B, TILE), lambda j: (0, j)),
            pl.BlockSpec((B, 1), lambda j: (0, 0)),
            pl.BlockSpec((B, TILE), lambda j: (0, 0)),
        ],
        out_specs=[
            pl.BlockSpec((B, TILE), lambda j: (0, j)),
            pl.BlockSpec((J0, 1, B), lambda j: (0, 0, 0)),
            pl.BlockSpec((J0, 1, B), lambda j: (0, 0, 0)),
        ],
        out_shape=[
            jax.ShapeDtypeStruct((b, vocab), jnp.float32),
            jax.ShapeDtypeStruct((J0, 1, B), jnp.float32),
            jax.ShapeDtypeStruct((J0, 1, B), jnp.int32),
        ],
        compiler_params=pltpu.CompilerParams(
            dimension_semantics=("arbitrary",)),
    )(logits, inv_col, flat0)

    probs, maxs_b, idxs_b = pl.pallas_call(
        functools.partial(_main_b, tile=TILE_B, vocab=vocab, col0=SC_COL0),
        grid=(NB,),
        in_specs=[
            pl.BlockSpec((B, TILE_B), lambda j: (0, j + J0 // 2)),
            pl.BlockSpec((B, TILE_B), lambda j: (0, j)),
            pl.BlockSpec((B, 1), lambda j: (0, 0)),
            pl.BlockSpec((8, 128), lambda j: (0, 0)),
        ],
        out_specs=[
            pl.BlockSpec((B, TILE_B), lambda j: (0, j + J0 // 2)),
            pl.BlockSpec((NB, 1, B), lambda j: (0, 0, 0)),
            pl.BlockSpec((NB, 1, B), lambda j: (0, 0, 0)),
        ],
        out_shape=[
            jax.ShapeDtypeStruct((b, vocab), jnp.float32),
            jax.ShapeDtypeStruct((NB, 1, B), jnp.float32),
            jax.ShapeDtypeStruct((NB, 1, B), jnp.int32),
        ],
        input_output_aliases={3: 0},
        compiler_params=pltpu.CompilerParams(
            dimension_semantics=("arbitrary",)),
    )(logits, sc_bits, inv_col, probs_a)

    maxs = jnp.concatenate([maxs_a, maxs_b], axis=0)
    idxs = jnp.concatenate([idxs_a, idxs_b], axis=0)

    idx = pl.pallas_call(
        _pass_idx,
        out_shape=jax.ShapeDtypeStruct((1, b), jnp.int32),
    )(maxs, idxs)

    idx_next = idx.reshape(b, 1).astype(jnp.int64)
    return (probs, idx_next)
